# baseline (speedup 1.0000x reference)
import functools

import jax
import jax.numpy as jnp
from jax import lax
from jax.experimental import pallas as pl
from jax.experimental.pallas import tpu as pltpu
from jax.experimental.pallas import tpu_sc as plsc

VOCAB = 1000000
MAXLEN = 200
EMBED_DIM = 64
BATCH = 4096

NC = 2
NS = 16
NW = NC * NS
ROWS_W = BATCH // NW
SUB = (128, 72)
LANES = 16
NBUF = 4
ROW_UNROLL = 8


def _body(idx_hbm, table_hbm, pos_hbm, out_hbm, idx_v, rows_v, pos_v, *sems):
    gsems = sems[:NBUF]
    osems = sems[NBUF:]
    wid = lax.axis_index("s") * NC + lax.axis_index("c")
    b0 = wid * ROWS_W

    pltpu.sync_copy(idx_hbm.at[pl.ds(b0, ROWS_W)], idx_v)
    pltpu.sync_copy(pos_hbm, pos_v)

    def gathers(i, s):
        cs = []
        sub_off = 0
        for n in SUB:
            cs.append(pltpu.make_async_copy(
                table_hbm.at[idx_v.at[i, pl.ds(sub_off, n)]],
                rows_v.at[s, pl.ds(sub_off, n)],
                gsems[s]))
            sub_off += n
        return cs

    def out_copy(i, s):
        return pltpu.make_async_copy(
            rows_v.at[s],
            out_hbm.at[b0 + i],
            osems[s])

    for s in range(NBUF - 1):
        for c in gathers(s, s):
            c.start()

    def chunk_body(i0, carry):
        for s in range(NBUF):
            i = i0 * NBUF + s
            sp = (s + NBUF - 1) % NBUF
            pf = i + NBUF - 1

            @pl.when(i > 0)
            def _():
                out_copy(i - 1, sp).wait()

            @pl.when(pf < ROWS_W)
            def _():
                for c in gathers(pf, sp):
                    c.start()

            for c in gathers(i, s):
                c.wait()

            def row_body(jj, c2):
                j = jj * ROW_UNROLL
                for r in range(ROW_UNROLL):
                    for c in range(EMBED_DIM // LANES):
                        sl = pl.ds(c * LANES, LANES)
                        rows_v[s, j + r, sl] = (
                            rows_v[s, j + r, sl] + pos_v[j + r, sl])
                return c2

            lax.fori_loop(0, MAXLEN // ROW_UNROLL, row_body, 0)
            out_copy(i, s).start()
        return carry

    lax.fori_loop(0, ROWS_W // NBUF, chunk_body, 0)
    out_copy(ROWS_W - 1, (ROWS_W - 1) % NBUF).wait()


def kernel(inputs, token_table, pos_table):
    idx = inputs.astype(jnp.int32)
    mesh = plsc.VectorSubcoreMesh(core_axis_name="c", subcore_axis_name="s")
    fn = functools.partial(
        pl.kernel,
        mesh=mesh,
        compiler_params=pltpu.CompilerParams(use_tc_tiling_on_sc=False),
        out_type=jax.ShapeDtypeStruct((BATCH, MAXLEN, EMBED_DIM), jnp.float32),
        scratch_types=[
            pltpu.VMEM((ROWS_W, MAXLEN), jnp.int32),
            pltpu.VMEM((NBUF, MAXLEN, EMBED_DIM), jnp.float32),
            pltpu.VMEM((MAXLEN, EMBED_DIM), jnp.float32),
        ] + [pltpu.SemaphoreType.DMA] * (2 * NBUF),
    )(_body)
    return fn(idx, token_table, pos_table)
